# Initial kernel scaffold; baseline (speedup 1.0000x reference)
import jax
import jax.numpy as jnp
from jax.experimental import pallas as pl


def kernel(coords, density, sh_coeffs):
    raise NotImplementedError("write your pallas kernel here")


# speedup vs baseline: 21.8470x; 21.8470x over previous
import functools
import jax
import jax.numpy as jnp
from jax import lax
from jax.experimental import pallas as pl
from jax.experimental.pallas import tpu as pltpu
from jax.experimental.pallas import tpu_sc as plsc

_RES = 128
_M = _RES * _RES * _RES
_N = 1048576
_NSH = 27
_ROW = 32

_NC = 2
_NS = 16
_NW = _NC * _NS
_PW = _N // _NW
_C = 128
_NCHUNK = _PW // _C
_G = _C // 16


def _take16(vec, idx):
    return lax.gather(
        vec, idx[:, None],
        dimension_numbers=lax.GatherDimensionNumbers(
            offset_dims=(), collapsed_slice_dims=(0,), start_index_map=(0,)),
        slice_sizes=(1,),
        mode=lax.GatherScatterMode.PROMISE_IN_BOUNDS)


def _sc_body(xs, ys, zs, table, dens_out, sh_out,
             cx, cy, cz, idxb, wb, rows, densb, shb, sem):
    wid = lax.axis_index("s") * _NC + lax.axis_index("c")
    base0 = wid * _PW

    lane = jnp.arange(16, dtype=jnp.int32)
    lane27 = lane * 27
    maxc = jnp.float32(_RES - 1)

    def chunk_body(c, carry):
        base = base0 + c * _C
        pltpu.sync_copy(xs.at[pl.ds(base, _C)], cx)
        pltpu.sync_copy(ys.at[pl.ds(base, _C)], cy)
        pltpu.sync_copy(zs.at[pl.ds(base, _C)], cz)

        def group_a(g, carry_a):
            p0 = g * 16

            def axis_prep(ref):
                v = ref[pl.ds(p0, 16)]
                norm = (v + 1.0) * 0.5
                vox = norm * jnp.float32(_RES)
                vox = jnp.minimum(jnp.maximum(vox, 0.0), maxc)
                i0 = vox.astype(jnp.int32)
                frac = vox - i0.astype(jnp.float32)
                off1 = jnp.minimum(i0 + 1, _RES - 1) - i0
                return i0, off1, frac

            x0, xo, dx = axis_prep(cx)
            y0, yo, dy = axis_prep(cy)
            z0, zo, dz = axis_prep(cz)

            b000 = (z0 * _RES + y0) * _RES + x0
            zoff = zo * (_RES * _RES)
            yoff = yo * _RES
            b100 = b000 + zoff
            b010 = b000 + yoff
            b110 = b100 + yoff
            idxs = (b000, b000 + zoff, b010, b110,
                    b000 + xo, b100 + xo, b010 + xo, b110 + xo)
            wx0 = 1.0 - dx
            wy0 = 1.0 - dy
            wz0 = 1.0 - dz
            a00 = wx0 * wy0
            a01 = wx0 * dy
            a10 = dx * wy0
            a11 = dx * dy
            ws = (a00 * wz0, a00 * dz, a01 * wz0, a01 * dz,
                  a10 * wz0, a10 * dz, a11 * wz0, a11 * dz)
            for k in range(8):
                idxb[k, pl.ds(p0, 16)] = idxs[k]
                wb[k, pl.ds(p0, 16)] = ws[k]
            return carry_a

        lax.fori_loop(0, _G, group_a, 0)

        descs = []
        for k in range(8):
            descs.append(pltpu.async_copy(
                table.at[idxb.at[k]], rows.at[pl.ds(k * _C, _C)], sem))
        for d in descs:
            d.wait()

        def group_c(g, carry_c):
            p0 = g * 16
            w_vecs = [wb[k, pl.ds(p0, 16)] for k in range(8)]
            for q in range(16):
                p = p0 + q
                sel = jnp.full((16,), q, dtype=jnp.int32)
                acc0 = jnp.zeros((16,), jnp.float32)
                acc1 = jnp.zeros((16,), jnp.float32)
                for k in range(8):
                    wk = _take16(w_vecs[k], sel)
                    r = k * _C + p
                    acc0 = acc0 + wk * rows[r, pl.ds(0, 16)]
                    acc1 = acc1 + wk * rows[r, pl.ds(16, 16)]
                plsc.store_scatter(
                    densb, [jnp.full((16,), p, dtype=jnp.int32)], acc0,
                    mask=lane == 0)
                plsc.store_scatter(
                    shb, [lane + (27 * p - 1)], acc0, mask=lane >= 1)
                plsc.store_scatter(
                    shb, [lane + (27 * p + 15)], acc1, mask=lane < 12)
            return carry_c

        lax.fori_loop(0, _G, group_c, 0)

        pltpu.sync_copy(densb, dens_out.at[pl.ds(base, _C)])
        pltpu.sync_copy(shb, sh_out.at[pl.ds(base * 27, _C * 27)])
        return carry

    lax.fori_loop(0, _NCHUNK, chunk_body, 0)


@jax.jit
def kernel(coords, density, sh_coeffs):
    table = jnp.concatenate(
        [density.reshape(_M, 1),
         sh_coeffs.reshape(_M, _NSH),
         jnp.zeros((_M, _ROW - 1 - _NSH), jnp.float32)], axis=1)
    xs = coords[:, 0]
    ys = coords[:, 1]
    zs = coords[:, 2]

    mesh = plsc.VectorSubcoreMesh(core_axis_name="c", subcore_axis_name="s")
    run = pl.kernel(
        _sc_body,
        out_type=(jax.ShapeDtypeStruct((_N,), jnp.float32),
                  jax.ShapeDtypeStruct((_N * _NSH,), jnp.float32)),
        mesh=mesh,
        compiler_params=pltpu.CompilerParams(
            needs_layout_passes=False, use_tc_tiling_on_sc=False),
        scratch_types=[
            pltpu.VMEM((_C,), jnp.float32),
            pltpu.VMEM((_C,), jnp.float32),
            pltpu.VMEM((_C,), jnp.float32),
            pltpu.VMEM((8, _C), jnp.int32),
            pltpu.VMEM((8, _C), jnp.float32),
            pltpu.VMEM((8 * _C, _ROW), jnp.float32),
            pltpu.VMEM((_C,), jnp.float32),
            pltpu.VMEM((_C * _NSH,), jnp.float32),
            pltpu.SemaphoreType.DMA,
        ],
    )
    dens, sh_flat = run(xs, ys, zs, table)
    return dens, sh_flat.reshape(_N, 3, 9)
